# baseline (speedup 1.0000x reference)
import functools

import jax
import jax.numpy as jnp
from jax import lax
from jax.experimental import pallas as pl
from jax.experimental.pallas import tpu as pltpu
from jax.experimental.pallas import tpu_sc as plsc

B, T, H, ENT = 4096, 50, 128, 11
BB = 256
G = B // BB
S = 8


def _dg(a, b, dims):
    return jax.lax.dot_general(a, b, (dims, ((), ())),
                               preferred_element_type=jnp.float32)


_NC, _NS = 2, 16
_NW = _NC * _NS
_PER_W = (B * T) // _NW
_CH = 3200
_NCHUNK = _PER_W // _CH


def _sc_transpose_body(tf_hbm, out_hbm, buf, obuf):
    wid = lax.axis_index("s") * _NC + lax.axis_index("c")
    iota = lax.iota(jnp.int32, 16)
    idx0 = iota * ENT

    def chunk(c, _):
        base = wid * _PER_W + c * _CH
        pltpu.sync_copy(tf_hbm.at[pl.ds(base * ENT, _CH * ENT)], buf)
        for i in range(ENT):
            def gath(g, _):
                idx = idx0 + (g * (16 * ENT) + i)
                obuf[i, pl.ds(g * 16, 16)] = plsc.load_gather(buf, [idx])
                return _
            lax.fori_loop(0, _CH // 16, gath, None)
        pltpu.sync_copy(obuf, out_hbm.at[:, pl.ds(base, _CH)])
        return _

    lax.fori_loop(0, _NCHUNK, chunk, None)


@functools.partial(
    pl.kernel,
    out_type=jax.ShapeDtypeStruct((ENT, B * T), jnp.float32),
    mesh=plsc.VectorSubcoreMesh(core_axis_name="c", subcore_axis_name="s"),
    scratch_types=[
        pltpu.VMEM((_CH * ENT,), jnp.float32),
        pltpu.VMEM((ENT, _CH), jnp.float32),
    ],
    compiler_params=pltpu.CompilerParams(needs_layout_passes=False),
)
def _sc_transpose(tf_hbm, out_hbm, buf, obuf):
    _sc_transpose_body(tf_hbm, out_hbm, buf, obuf)


def _attack_kernel(stim_ref, tf_ref, st_ref, went_ref, bent_ref, wsty_ref,
                   wkey_ref, scores_ref, logits_ref, atn_ref, arg_ref):
    stim = stim_ref[...]

    logits = _dg(stim, wsty_ref[...], ((1,), (0,)))
    k = _dg(stim, wkey_ref[...], ((1,), (0,)))
    k1 = k[:, :H]
    k2 = k[:, H:]

    iota3 = jax.lax.broadcasted_iota(jnp.int32, (BB, 3), 1)
    m3 = jnp.max(logits, axis=1, keepdims=True)
    atn = jnp.min(jnp.where(logits >= m3, iota3, 3), axis=1,
                  keepdims=True)
    s1_all = _dg(k1, st_ref[...], ((1,), (1,)))
    s1 = jnp.sum(jnp.where(iota3 == atn, s1_all, 0.0), axis=1,
                 keepdims=True)
    s1 = s1 + _dg(k2, bent_ref[...], ((1,), (0,)))

    targ_t = _dg(went_ref[...], tf_ref[...], ((1,), (0,)))

    lane = jax.lax.broadcasted_iota(jnp.int32, (S, S * T), 1)
    row = jax.lax.broadcasted_iota(jnp.int32, (S, S * T), 0)
    grp = lane // T
    mask = grp == row
    t_lane = lane - grp * T
    inv16 = jnp.float32(1.0 / 16.0)

    score_parts = []
    arg_parts = []
    for s in range(BB // S):
        cross = _dg(k2[s * S:(s + 1) * S, :],
                    targ_t[:, s * S * T:(s + 1) * S * T],
                    ((1,), (0,)))
        cross = (cross + s1[s * S:(s + 1) * S, :]) * inv16
        score_parts.extend(cross[i:i + 1, i * T:(i + 1) * T]
                           for i in range(S))
        neg = jnp.where(mask, cross, -jnp.inf)
        cmax = jnp.max(neg, axis=1, keepdims=True)
        arg_parts.append(jnp.min(jnp.where(neg >= cmax, t_lane, T),
                                 axis=1, keepdims=True))

    scores_ref[...] = jnp.concatenate(score_parts, axis=0)
    logits_ref[...] = logits
    atn_ref[...] = atn
    arg_ref[...] = jnp.concatenate(arg_parts, axis=0)


def kernel(stim, targFeats, styleTable, W_ent, b_ent, W_style, W_key):
    tf_t = _sc_transpose(targFeats.reshape(B * T * ENT))
    bent_col = b_ent.reshape(H, 1)
    went_t = W_ent.T
    full = lambda i: (0, 0)
    row = lambda i: (i, 0)
    scores, logits, atn, arg = pl.pallas_call(
        _attack_kernel,
        grid=(G,),
        compiler_params=pltpu.CompilerParams(
            dimension_semantics=("arbitrary",)),
        in_specs=[
            pl.BlockSpec((BB, 2 * H), row),
            pl.BlockSpec((ENT, BB * T), lambda i: (0, i)),
            pl.BlockSpec((3, H), full),
            pl.BlockSpec((H, ENT), full),
            pl.BlockSpec((H, 1), full),
            pl.BlockSpec((2 * H, 3), full),
            pl.BlockSpec((2 * H, 2 * H), full),
        ],
        out_specs=[
            pl.BlockSpec((BB, T), row),
            pl.BlockSpec((BB, 3), row),
            pl.BlockSpec((BB, 1), row),
            pl.BlockSpec((BB, 1), row),
        ],
        out_shape=[
            jax.ShapeDtypeStruct((B, T), jnp.float32),
            jax.ShapeDtypeStruct((B, 3), jnp.float32),
            jax.ShapeDtypeStruct((B, 1), jnp.int32),
            jax.ShapeDtypeStruct((B, 1), jnp.int32),
        ],
    )(stim, tf_t, styleTable, went_t, bent_col, W_style, W_key)
    return (scores, logits, atn.reshape(B), arg.reshape(B))


# speedup vs baseline: 1.0094x; 1.0094x over previous
import functools

import jax
import jax.numpy as jnp
from jax import lax
from jax.experimental import pallas as pl
from jax.experimental.pallas import tpu as pltpu
from jax.experimental.pallas import tpu_sc as plsc

B, T, H, ENT = 4096, 50, 128, 11
BB = 256
G = B // BB
S = 8


def _dg(a, b, dims):
    return jax.lax.dot_general(a, b, (dims, ((), ())),
                               preferred_element_type=jnp.float32)


_NC, _NS = 2, 16
_NW = _NC * _NS
_PER_W = (B * T) // _NW
_CH = 3200
_NCHUNK = _PER_W // _CH


def _sc_transpose_body(tf_hbm, out_hbm, buf, obuf):
    wid = lax.axis_index("s") * _NC + lax.axis_index("c")
    iota = lax.iota(jnp.int32, 16)
    idx0 = iota * ENT

    def chunk(c, _):
        base = wid * _PER_W + c * _CH
        pltpu.sync_copy(tf_hbm.at[pl.ds(base * ENT, _CH * ENT)], buf)

        def gath(g, _):
            gbase = idx0 + g * (16 * ENT)
            for i in range(ENT):
                obuf[i, pl.ds(g * 16, 16)] = plsc.load_gather(buf, [gbase + i])
            return _
        lax.fori_loop(0, _CH // 16, gath, None, unroll=2)
        pltpu.sync_copy(obuf, out_hbm.at[:, pl.ds(base, _CH)])
        return _

    lax.fori_loop(0, _NCHUNK, chunk, None)


@functools.partial(
    pl.kernel,
    out_type=jax.ShapeDtypeStruct((ENT, B * T), jnp.float32),
    mesh=plsc.VectorSubcoreMesh(core_axis_name="c", subcore_axis_name="s"),
    scratch_types=[
        pltpu.VMEM((_CH * ENT,), jnp.float32),
        pltpu.VMEM((ENT, _CH), jnp.float32),
    ],
    compiler_params=pltpu.CompilerParams(needs_layout_passes=False),
)
def _sc_transpose(tf_hbm, out_hbm, buf, obuf):
    _sc_transpose_body(tf_hbm, out_hbm, buf, obuf)


def _attack_kernel(stim_ref, tf_ref, st_ref, went_ref, bent_ref, wsty_ref,
                   wkey_ref, scores_ref, logits_ref, atn_ref, arg_ref):
    stim = stim_ref[...]

    logits = _dg(stim, wsty_ref[...], ((1,), (0,)))
    k = _dg(stim, wkey_ref[...], ((1,), (0,)))
    k1 = k[:, :H]
    k2 = k[:, H:]

    iota3 = jax.lax.broadcasted_iota(jnp.int32, (BB, 3), 1)
    m3 = jnp.max(logits, axis=1, keepdims=True)
    atn = jnp.min(jnp.where(logits >= m3, iota3, 3), axis=1,
                  keepdims=True)
    s1_all = _dg(k1, st_ref[...], ((1,), (1,)))
    s1 = jnp.sum(jnp.where(iota3 == atn, s1_all, 0.0), axis=1,
                 keepdims=True)
    s1 = s1 + _dg(k2, bent_ref[...], ((1,), (0,)))

    targ_t = _dg(went_ref[...], tf_ref[...], ((1,), (0,)))

    lane = jax.lax.broadcasted_iota(jnp.int32, (S, S * T), 1)
    row = jax.lax.broadcasted_iota(jnp.int32, (S, S * T), 0)
    grp = lane // T
    mask = grp == row
    t_lane = lane - grp * T
    inv16 = jnp.float32(1.0 / 16.0)

    score_parts = []
    arg_parts = []
    for s in range(BB // S):
        cross = _dg(k2[s * S:(s + 1) * S, :],
                    targ_t[:, s * S * T:(s + 1) * S * T],
                    ((1,), (0,)))
        cross = (cross + s1[s * S:(s + 1) * S, :]) * inv16
        score_parts.extend(cross[i:i + 1, i * T:(i + 1) * T]
                           for i in range(S))
        neg = jnp.where(mask, cross, -jnp.inf)
        cmax = jnp.max(neg, axis=1, keepdims=True)
        arg_parts.append(jnp.min(jnp.where(neg >= cmax, t_lane, T),
                                 axis=1, keepdims=True))

    scores_ref[...] = jnp.concatenate(score_parts, axis=0)
    logits_ref[...] = logits
    atn_ref[...] = atn
    arg_ref[...] = jnp.concatenate(arg_parts, axis=0)


def kernel(stim, targFeats, styleTable, W_ent, b_ent, W_style, W_key):
    tf_t = _sc_transpose(targFeats.reshape(B * T * ENT))
    bent_col = b_ent.reshape(H, 1)
    went_t = W_ent.T
    full = lambda i: (0, 0)
    row = lambda i: (i, 0)
    scores, logits, atn, arg = pl.pallas_call(
        _attack_kernel,
        grid=(G,),
        compiler_params=pltpu.CompilerParams(
            dimension_semantics=("arbitrary",)),
        in_specs=[
            pl.BlockSpec((BB, 2 * H), row),
            pl.BlockSpec((ENT, BB * T), lambda i: (0, i)),
            pl.BlockSpec((3, H), full),
            pl.BlockSpec((H, ENT), full),
            pl.BlockSpec((H, 1), full),
            pl.BlockSpec((2 * H, 3), full),
            pl.BlockSpec((2 * H, 2 * H), full),
        ],
        out_specs=[
            pl.BlockSpec((BB, T), row),
            pl.BlockSpec((BB, 3), row),
            pl.BlockSpec((BB, 1), row),
            pl.BlockSpec((BB, 1), row),
        ],
        out_shape=[
            jax.ShapeDtypeStruct((B, T), jnp.float32),
            jax.ShapeDtypeStruct((B, 3), jnp.float32),
            jax.ShapeDtypeStruct((B, 1), jnp.int32),
            jax.ShapeDtypeStruct((B, 1), jnp.int32),
        ],
    )(stim, tf_t, styleTable, went_t, bent_col, W_style, W_key)
    return (scores, logits, atn.reshape(B), arg.reshape(B))


# speedup vs baseline: 1.5053x; 1.4913x over previous
import jax
import jax.numpy as jnp
from jax.experimental import pallas as pl
from jax.experimental.pallas import tpu as pltpu

B, T, H, ENT = 4096, 50, 128, 11
BB = 256
G = B // BB
S = 8


def _dg(a, b, dims):
    return jax.lax.dot_general(a, b, (dims, ((), ())),
                               preferred_element_type=jnp.float32)


def _attack_kernel(stim_ref, tf_ref, st_ref, went_ref, bent_ref, wsty_ref,
                   wkey_ref, scores_ref, logits_ref, atn_ref, arg_ref):
    stim = stim_ref[...]

    logits = _dg(stim, wsty_ref[...], ((1,), (0,)))
    k = _dg(stim, wkey_ref[...], ((1,), (0,)))
    k1 = k[:, :H]
    k2 = k[:, H:]

    iota3 = jax.lax.broadcasted_iota(jnp.int32, (BB, 3), 1)
    m3 = jnp.max(logits, axis=1, keepdims=True)
    atn = jnp.min(jnp.where(logits >= m3, iota3, 3), axis=1,
                  keepdims=True)
    s1_all = _dg(k1, st_ref[...], ((1,), (1,)))
    s1 = jnp.sum(jnp.where(iota3 == atn, s1_all, 0.0), axis=1,
                 keepdims=True)
    s1 = s1 + _dg(k2, bent_ref[...], ((1,), (0,)))

    targ_t = _dg(went_ref[...], tf_ref[...], ((1,), (0,)))

    lane = jax.lax.broadcasted_iota(jnp.int32, (S, S * T), 1)
    row = jax.lax.broadcasted_iota(jnp.int32, (S, S * T), 0)
    grp = lane // T
    mask = grp == row
    t_lane = lane - grp * T
    inv16 = jnp.float32(1.0 / 16.0)

    score_parts = []
    arg_parts = []
    for s in range(BB // S):
        cross = _dg(k2[s * S:(s + 1) * S, :],
                    targ_t[:, s * S * T:(s + 1) * S * T],
                    ((1,), (0,)))
        cross = (cross + s1[s * S:(s + 1) * S, :]) * inv16
        score_parts.extend(cross[i:i + 1, i * T:(i + 1) * T]
                           for i in range(S))
        neg = jnp.where(mask, cross, -jnp.inf)
        cmax = jnp.max(neg, axis=1, keepdims=True)
        arg_parts.append(jnp.min(jnp.where(neg >= cmax, t_lane, T),
                                 axis=1, keepdims=True))

    scores_ref[...] = jnp.concatenate(score_parts, axis=0)
    logits_ref[...] = logits
    atn_ref[...] = atn
    arg_ref[...] = jnp.concatenate(arg_parts, axis=0)


def kernel(stim, targFeats, styleTable, W_ent, b_ent, W_style, W_key):
    tf_t = targFeats.reshape(B * T, ENT).T
    bent_col = b_ent.reshape(H, 1)
    went_t = W_ent.T
    full = lambda i: (0, 0)
    row = lambda i: (i, 0)
    scores, logits, atn, arg = pl.pallas_call(
        _attack_kernel,
        grid=(G,),
        compiler_params=pltpu.CompilerParams(
            dimension_semantics=("arbitrary",)),
        in_specs=[
            pl.BlockSpec((BB, 2 * H), row),
            pl.BlockSpec((ENT, BB * T), lambda i: (0, i)),
            pl.BlockSpec((3, H), full),
            pl.BlockSpec((H, ENT), full),
            pl.BlockSpec((H, 1), full),
            pl.BlockSpec((2 * H, 3), full),
            pl.BlockSpec((2 * H, 2 * H), full),
        ],
        out_specs=[
            pl.BlockSpec((BB, T), row),
            pl.BlockSpec((BB, 3), row),
            pl.BlockSpec((BB, 1), row),
            pl.BlockSpec((BB, 1), row),
        ],
        out_shape=[
            jax.ShapeDtypeStruct((B, T), jnp.float32),
            jax.ShapeDtypeStruct((B, 3), jnp.float32),
            jax.ShapeDtypeStruct((B, 1), jnp.int32),
            jax.ShapeDtypeStruct((B, 1), jnp.int32),
        ],
    )(stim, tf_t, styleTable, went_t, bent_col, W_style, W_key)
    return (scores, logits, atn.reshape(B), arg.reshape(B))
